# baseline (speedup 1.0000x reference)
import functools

import jax
import jax.numpy as jnp
import numpy as np
from jax import lax
from jax.experimental import pallas as pl
from jax.experimental.pallas import tpu as pltpu

_TINY = np.float32(np.finfo(np.float32).tiny)
_DIFF = np.float32(1.0) - _TINY
_ONE_BITS = np.uint32(0x3F800000)
_INT_MAX = np.int32(np.iinfo(np.int32).max)


def _rotl(x, r):
    return (x << np.uint32(r)) | (x >> np.uint32(32 - r))


def _threefry2x32_xor(k1, k2, x0, x1):
    ks2 = k1 ^ k2 ^ np.uint32(0x1BD11BDA)

    def rounds(x0, x1, rots):
        for r in rots:
            x0 = x0 + x1
            x1 = _rotl(x1, r)
            x1 = x0 ^ x1
        return x0, x1

    x0 = x0 + k1
    x1 = x1 + k2
    x0, x1 = rounds(x0, x1, (13, 15, 26, 6))
    x0 = x0 + k2
    x1 = x1 + ks2 + np.uint32(1)
    x0, x1 = rounds(x0, x1, (17, 29, 16, 24))
    x0 = x0 + ks2
    x1 = x1 + k1 + np.uint32(2)
    x0, x1 = rounds(x0, x1, (13, 15, 26, 6))
    x0 = x0 + k1
    x1 = x1 + k2 + np.uint32(3)
    x0, x1 = rounds(x0, x1, (17, 29, 16, 24))
    x0 = x0 + k2
    x1 = x1 + ks2 + np.uint32(4)
    x0, x1 = rounds(x0, x1, (13, 15, 26, 6))
    x0 = x0 + ks2
    x1 = x1 + k1 + np.uint32(5)
    return x0 ^ x1


def _body(kd_ref, logits_ref, out_ref, maxref, idxref, *, nblk, V, Vb, B):
    i = pl.program_id(0)
    k1 = kd_ref[0]
    k2 = kd_ref[1]

    rows = lax.broadcasted_iota(jnp.uint32, (B, Vb), 0)
    lanes = lax.broadcasted_iota(jnp.uint32, (B, Vb), 1)
    j = rows * jnp.uint32(V) + (lanes + jnp.uint32(Vb) * i.astype(jnp.uint32))
    bits = _threefry2x32_xor(k1, k2, jnp.zeros_like(j), j)

    fb = (bits >> jnp.uint32(9)) | _ONE_BITS
    floats = lax.bitcast_convert_type(fb, jnp.float32) - jnp.float32(1.0)
    u = jnp.maximum(_TINY, floats * _DIFF + _TINY)
    g = -jnp.log(-jnp.log(u))

    vids = lax.broadcasted_iota(jnp.int32, (B, Vb), 1) + i * Vb
    cand = logits_ref[...] + g
    cand = jnp.where(vids < V, cand, -jnp.inf)

    @pl.when(i == 0)
    def _():
        maxref[...] = cand
        idxref[...] = vids

    @pl.when(i > 0)
    def _():
        old = maxref[...]
        take = cand > old
        maxref[...] = jnp.where(take, cand, old)
        idxref[...] = jnp.where(take, vids, idxref[...])

    @pl.when(i == nblk - 1)
    def _():
        m = jnp.max(maxref[...], axis=1, keepdims=True)
        sel = jnp.where(maxref[...] == m, idxref[...], _INT_MAX)
        out_ref[...] = jnp.broadcast_to(
            jnp.min(sel, axis=1, keepdims=True), (B, 128)
        )


def _gumbel_argmax(logits, key_data, Vb=8192):
    B, V = logits.shape
    nblk = pl.cdiv(V, Vb)
    out = pl.pallas_call(
        functools.partial(_body, nblk=nblk, V=V, Vb=Vb, B=B),
        grid=(nblk,),
        in_specs=[
            pl.BlockSpec(memory_space=pltpu.SMEM),
            pl.BlockSpec((B, Vb), lambda i: (0, i)),
        ],
        out_specs=pl.BlockSpec((B, 128), lambda i: (0, 0)),
        out_shape=jax.ShapeDtypeStruct((B, 128), jnp.int32),
        scratch_shapes=[
            pltpu.VMEM((B, Vb), jnp.float32),
            pltpu.VMEM((B, Vb), jnp.int32),
        ],
    )(key_data, logits)
    return out[:, 0]


def kernel(logits, seed, num_samples):
    B, V = logits.shape
    kd = jax.random.key_data(jax.random.key(seed)).astype(jnp.uint32)
    samples = _gumbel_argmax(logits, kd).reshape(1, B)
    return samples + jnp.asarray(num_samples - 1, dtype=samples.dtype)


# speedup vs baseline: 1.7175x; 1.7175x over previous
import functools

import jax
import jax.numpy as jnp
import numpy as np
from jax import lax
from jax.experimental import pallas as pl
from jax.experimental.pallas import tpu as pltpu

_TINY = np.float32(np.finfo(np.float32).tiny)
_DIFF = np.float32(1.0) - _TINY
_ONE_BITS = np.uint32(0x3F800000)
_INT_MAX = np.int32(np.iinfo(np.int32).max)
_WC = 128


def _rotl(x, r):
    return (x << np.uint32(r)) | (x >> np.uint32(32 - r))


def _rounds(x0, x1, rots):
    for r in rots:
        x0 = x0 + x1
        x1 = _rotl(x1, r)
        x1 = x0 ^ x1
    return x0, x1


def _threefry2x32_xor(k1, k2, ks2, x1):
    x0, x1 = _rounds(k1, x1, (13, 15, 26, 6))
    x0 = x0 + k2
    x1 = x1 + (ks2 + np.uint32(1))
    x0, x1 = _rounds(x0, x1, (17, 29, 16, 24))
    x0 = x0 + ks2
    x1 = x1 + (k1 + np.uint32(2))
    x0, x1 = _rounds(x0, x1, (13, 15, 26, 6))
    x0 = x0 + k1
    x1 = x1 + (k2 + np.uint32(3))
    x0, x1 = _rounds(x0, x1, (17, 29, 16, 24))
    x0 = x0 + k2
    x1 = x1 + (ks2 + np.uint32(4))
    x0, x1 = _rounds(x0, x1, (13, 15, 26, 6))
    x0 = x0 + ks2
    x1 = x1 + (k1 + np.uint32(5))
    return x0 ^ x1


def _body(kd_ref, logits_ref, out_ref, maxref, idxref, *, nblk, V, Vb, B):
    i = pl.program_id(0)
    k1 = kd_ref[0]
    k2 = kd_ref[1]
    ks2 = k1 ^ k2 ^ np.uint32(0x1BD11BDA)
    nch = Vb // _WC

    rows = lax.broadcasted_iota(jnp.uint32, (B, _WC), 0)
    lanes_u = lax.broadcasted_iota(jnp.uint32, (B, _WC), 1)
    lanes_i = lax.broadcasted_iota(jnp.int32, (B, _WC), 1)
    row_lane = rows * jnp.uint32(V) + lanes_u

    first = i == 0
    maxacc = jnp.where(first, -jnp.inf, maxref[...])
    idxacc = jnp.where(first, 0, idxref[...])
    base = i * Vb

    for c in range(nch):
        logits_c = logits_ref[:, c * _WC:(c + 1) * _WC]
        voff = base + c * _WC
        x1 = row_lane + (voff.astype(jnp.uint32) + k2)
        bits = _threefry2x32_xor(k1, k2, ks2, x1)

        fb = (bits >> jnp.uint32(9)) | _ONE_BITS
        floats = lax.bitcast_convert_type(fb, jnp.float32) - jnp.float32(1.0)
        u = jnp.maximum(_TINY, floats * _DIFF + _TINY)
        g = -jnp.log(-jnp.log(u))

        cand = logits_c + g
        cand = jnp.where(lanes_i < V - voff, cand, -jnp.inf)
        take = cand > maxacc
        maxacc = jnp.where(take, cand, maxacc)
        idxacc = jnp.where(take, i * nch + c, idxacc)

    maxref[...] = maxacc
    idxref[...] = idxacc

    @pl.when(i == nblk - 1)
    def _():
        mv = maxref[...]
        vfull = idxref[...] * _WC + lanes_i
        m = jnp.max(mv, axis=1, keepdims=True)
        sel = jnp.where(mv == m, vfull, _INT_MAX)
        out_ref[...] = jnp.broadcast_to(
            jnp.min(sel, axis=1, keepdims=True), (B, 128)
        )


def _gumbel_argmax(logits, key_data, Vb=2048):
    B, V = logits.shape
    nblk = pl.cdiv(V, Vb)
    out = pl.pallas_call(
        functools.partial(_body, nblk=nblk, V=V, Vb=Vb, B=B),
        grid=(nblk,),
        in_specs=[
            pl.BlockSpec(memory_space=pltpu.SMEM),
            pl.BlockSpec((B, Vb), lambda i: (0, i)),
        ],
        out_specs=pl.BlockSpec((B, 128), lambda i: (0, 0)),
        out_shape=jax.ShapeDtypeStruct((B, 128), jnp.int32),
        scratch_shapes=[
            pltpu.VMEM((B, 128), jnp.float32),
            pltpu.VMEM((B, 128), jnp.int32),
        ],
    )(key_data, logits)
    return out[:, 0]


def kernel(logits, seed, num_samples):
    B, V = logits.shape
    kd = jax.random.key_data(jax.random.key(seed)).astype(jnp.uint32)
    samples = _gumbel_argmax(logits, kd).reshape(1, B)
    return samples + jnp.asarray(num_samples - 1, dtype=samples.dtype)
